# baseline (speedup 1.0000x reference)
import functools

import jax
import jax.numpy as jnp
from jax import lax
from jax.experimental import pallas as pl
from jax.experimental.pallas import tpu as pltpu
from jax.experimental.pallas import tpu_sc as plsc


def _sc_gather_pack(jdx2d, hb2d, w32c, N, D):
    V2, _ = w32c.shape
    NC, NS, LN = 2, 16, 16
    NW = NC * NS
    CH = 128
    PR = CH // 2
    per_w = N // NW
    n_ch = per_w // CH

    mesh = plsc.VectorSubcoreMesh(core_axis_name="c", subcore_axis_name="s")

    @functools.partial(
        pl.kernel,
        mesh=mesh,
        out_type=jax.ShapeDtypeStruct((N // 2, D), jnp.int32),
        scratch_types=[
            pltpu.VMEM((n_ch, CH), jnp.int32),
            pltpu.VMEM((n_ch, CH), jnp.int32),
            pltpu.VMEM((2 * CH, D), jnp.int32),
            pltpu.VMEM((2 * PR, D), jnp.int32),
            pltpu.SemaphoreType.DMA,
            pltpu.SemaphoreType.DMA,
            pltpu.SemaphoreType.DMA,
            pltpu.SemaphoreType.DMA,
        ],
    )
    def emb(jdx_hbm, hb_hbm, w_hbm, out_hbm, jdx_v, hb_v, units_v, pairs_v,
            gsem0, gsem1, osem0, osem1):
        wid = lax.axis_index("s") * NC + lax.axis_index("c")
        pltpu.sync_copy(jdx_hbm.at[pl.ds(wid * n_ch, n_ch)], jdx_v)
        pltpu.sync_copy(hb_hbm.at[pl.ds(wid * n_ch, n_ch)], hb_v)
        pbase = wid * (per_w // 2)

        def fire_gather(r, x, gsem):
            pltpu.async_copy(
                w_hbm.at[jdx_v.at[r]],
                units_v.at[pl.ds(x * CH, CH)],
                gsem,
            )

        def drain_gather(x, gsem):
            pltpu.make_async_copy(
                w_hbm.at[pl.ds(0, CH)],
                units_v.at[pl.ds(x * CH, CH)],
                gsem,
            ).wait()

        def fire_out(r, x, osem):
            pltpu.async_copy(
                pairs_v.at[pl.ds(x * PR, PR)],
                out_hbm.at[pl.ds(pbase + r * PR, PR)],
                osem,
            )

        def drain_out(x, osem):
            pltpu.make_async_copy(
                pairs_v.at[pl.ds(x * PR, PR)],
                out_hbm.at[pl.ds(pbase, PR)],
                osem,
            ).wait()

        dnums = lax.GatherDimensionNumbers(
            offset_dims=(), collapsed_slice_dims=(0,), start_index_map=(0,))

        def bcast(v, j):
            idx = jnp.full((LN, 1), j, jnp.int32)
            return lax.gather(v, idx, dnums, (1,),
                              mode=lax.GatherScatterMode.PROMISE_IN_BOUNDS)

        def transform(r, x):
            @plsc.parallel_loop(0, CH // LN, unroll=3)
            def _(q):
                hvec = hb_v[r, pl.ds(q * LN, LN)] * 16
                for p in range(LN // 2):
                    k = q * (LN // 2) + p
                    s0 = bcast(hvec, 2 * p)
                    s1 = bcast(hvec, 2 * p + 1)
                    u0 = x * CH + 2 * k
                    for t in range(D // LN):
                        a = units_v[u0, pl.ds(t * LN, LN)]
                        b = units_v[u0 + 1, pl.ds(t * LN, LN)]
                        a = lax.shift_right_logical(a, s0)
                        b = lax.shift_right_logical(b, s1)
                        e = (a & 0xFFFF) | lax.shift_left(b, 16)
                        pairs_v[x * PR + k, pl.ds(t * LN, LN)] = e

        fire_gather(0, 0, gsem0)

        def round2(i, carry):
            r0 = 2 * i
            drain_gather(0, gsem0)

            @pl.when(r0 + 1 < n_ch)
            def _():
                fire_gather(r0 + 1, 1, gsem1)

            @pl.when(i >= 1)
            def _():
                drain_out(0, osem0)

            transform(r0, 0)
            fire_out(r0, 0, osem0)

            drain_gather(1, gsem1)

            @pl.when(r0 + 2 < n_ch)
            def _():
                fire_gather(r0 + 2, 0, gsem0)

            @pl.when(i >= 1)
            def _():
                drain_out(1, osem1)

            transform(r0 + 1, 1)
            fire_out(r0 + 1, 1, osem1)
            return carry

        lax.fori_loop(0, n_ch // 2, round2, 0)
        drain_out(0, osem0)
        drain_out(1, osem1)

    return emb(jdx2d, hb2d, w32c)


def _tc_retile(packed, B, L, D):
    NB = 256

    def body(in_ref, out_ref):
        y = pltpu.bitcast(in_ref[...], jnp.bfloat16)
        out_ref[...] = y.reshape(NB, L, D)

    return pl.pallas_call(
        body,
        grid=(B // NB,),
        in_specs=[pl.BlockSpec((NB * L // 2, D), lambda i: (i, 0))],
        out_specs=pl.BlockSpec((NB, L, D), lambda i: (i, 0, 0)),
        out_shape=jax.ShapeDtypeStruct((B, L, D), jnp.bfloat16),
    )(packed)


def _tc_pack_table(weight, V, D):
    RB = 10000

    def body(in_ref, out_ref):
        out_ref[...] = pltpu.bitcast(in_ref[...], jnp.int32)

    return pl.pallas_call(
        body,
        grid=(V // RB,),
        in_specs=[pl.BlockSpec((RB, D), lambda i: (i, 0))],
        out_specs=pl.BlockSpec((RB // 2, D), lambda i: (i, 0)),
        out_shape=jax.ShapeDtypeStruct((V // 2, D), jnp.int32),
    )(weight)


def kernel(indices, weight):
    B, L = indices.shape
    V, D = weight.shape
    N = B * L

    w32c = _tc_pack_table(weight, V, D)

    flat = indices.reshape(N // 128, 128)
    jdx2d = flat >> 1
    hb2d = flat & 1

    packed = _sc_gather_pack(jdx2d, hb2d, w32c, N, D)
    return _tc_retile(packed, B, L, D)


# speedup vs baseline: 1.4251x; 1.4251x over previous
import functools

import jax
import jax.numpy as jnp
from jax import lax
from jax.experimental import pallas as pl
from jax.experimental.pallas import tpu as pltpu
from jax.experimental.pallas import tpu_sc as plsc


def _sc_gather_pack(jdx2d, hb2d, w32c, N, D):
    V2, _ = w32c.shape
    NC, NS, LN = 2, 16, 16
    NW = NC * NS
    CH = 128
    PR = CH // 2
    per_w = N // NW
    n_ch = per_w // CH

    mesh = plsc.VectorSubcoreMesh(core_axis_name="c", subcore_axis_name="s")

    @functools.partial(
        pl.kernel,
        mesh=mesh,
        out_type=jax.ShapeDtypeStruct((N // 2, D), jnp.int32),
        scratch_types=[
            pltpu.VMEM((n_ch, CH), jnp.int32),
            pltpu.VMEM((n_ch, CH), jnp.int32),
            pltpu.VMEM((2 * CH, D), jnp.int32),
            pltpu.VMEM((2 * PR, D), jnp.int32),
            pltpu.SemaphoreType.DMA,
            pltpu.SemaphoreType.DMA,
            pltpu.SemaphoreType.DMA,
            pltpu.SemaphoreType.DMA,
        ],
    )
    def emb(jdx_hbm, hb_hbm, w_hbm, out_hbm, jdx_v, hb_v, units_v, pairs_v,
            gsem0, gsem1, osem0, osem1):
        wid = lax.axis_index("s") * NC + lax.axis_index("c")
        pltpu.sync_copy(jdx_hbm.at[pl.ds(wid * n_ch, n_ch)], jdx_v)
        pltpu.sync_copy(hb_hbm.at[pl.ds(wid * n_ch, n_ch)], hb_v)
        pbase = wid * (per_w // 2)

        def fire_gather(r, x, gsem):
            pltpu.async_copy(
                w_hbm.at[jdx_v.at[r]],
                units_v.at[pl.ds(x * CH, CH)],
                gsem,
            )

        def drain_gather(x, gsem):
            pltpu.make_async_copy(
                w_hbm.at[pl.ds(0, CH)],
                units_v.at[pl.ds(x * CH, CH)],
                gsem,
            ).wait()

        def fire_out(r, x, osem):
            pltpu.async_copy(
                pairs_v.at[pl.ds(x * PR, PR)],
                out_hbm.at[pl.ds(pbase + r * PR, PR)],
                osem,
            )

        def drain_out(x, osem):
            pltpu.make_async_copy(
                pairs_v.at[pl.ds(x * PR, PR)],
                out_hbm.at[pl.ds(pbase, PR)],
                osem,
            ).wait()

        dnums = lax.GatherDimensionNumbers(
            offset_dims=(), collapsed_slice_dims=(0,), start_index_map=(0,))

        def bcast(v, j):
            idx = jnp.full((LN, 1), j, jnp.int32)
            return lax.gather(v, idx, dnums, (1,),
                              mode=lax.GatherScatterMode.PROMISE_IN_BOUNDS)

        def transform(r, x):
            @plsc.parallel_loop(0, CH // LN, unroll=2)
            def _(q):
                hvec = hb_v[r, pl.ds(q * LN, LN)] * 16
                for p in range(LN // 2):
                    k = q * (LN // 2) + p
                    s0 = bcast(hvec, 2 * p)
                    s1 = bcast(hvec, 2 * p + 1)
                    u0 = x * CH + 2 * k
                    for t in range(D // LN):
                        a = units_v[u0, pl.ds(t * LN, LN)]
                        b = units_v[u0 + 1, pl.ds(t * LN, LN)]
                        a = lax.shift_right_logical(a, s0)
                        b = lax.shift_right_logical(b, s1)
                        e = (a & 0xFFFF) | lax.shift_left(b, 16)
                        pairs_v[x * PR + k, pl.ds(t * LN, LN)] = e

        fire_gather(0, 0, gsem0)

        def round2(i, carry):
            r0 = 2 * i
            drain_gather(0, gsem0)

            @pl.when(r0 + 1 < n_ch)
            def _():
                fire_gather(r0 + 1, 1, gsem1)

            @pl.when(i >= 1)
            def _():
                drain_out(0, osem0)

            transform(r0, 0)
            fire_out(r0, 0, osem0)

            drain_gather(1, gsem1)

            @pl.when(r0 + 2 < n_ch)
            def _():
                fire_gather(r0 + 2, 0, gsem0)

            @pl.when(i >= 1)
            def _():
                drain_out(1, osem1)

            transform(r0 + 1, 1)
            fire_out(r0 + 1, 1, osem1)
            return carry

        lax.fori_loop(0, n_ch // 2, round2, 0)
        drain_out(0, osem0)
        drain_out(1, osem1)

    return emb(jdx2d, hb2d, w32c)


def _tc_retile(packed, B, L, D):
    NB = 128

    def body(in_ref, out_ref):
        y = pltpu.bitcast(in_ref[...], jnp.bfloat16)
        out_ref[...] = y.reshape(NB, L, D)

    return pl.pallas_call(
        body,
        grid=(B // NB,),
        in_specs=[pl.BlockSpec((NB * L // 2, D), lambda i: (i, 0))],
        out_specs=pl.BlockSpec((NB, L, D), lambda i: (i, 0, 0)),
        out_shape=jax.ShapeDtypeStruct((B, L, D), jnp.bfloat16),
    )(packed)


def _tc_pack_table(weight, V, D):
    RB = 10000

    def body(in_ref, out_ref):
        out_ref[...] = pltpu.bitcast(in_ref[...], jnp.int32)

    return pl.pallas_call(
        body,
        grid=(V // RB,),
        in_specs=[pl.BlockSpec((RB, D), lambda i: (i, 0))],
        out_specs=pl.BlockSpec((RB // 2, D), lambda i: (i, 0)),
        out_shape=jax.ShapeDtypeStruct((V // 2, D), jnp.int32),
    )(weight)


def kernel(indices, weight):
    B, L = indices.shape
    V, D = weight.shape
    N = B * L

    w32c = _tc_pack_table(weight, V, D)

    flat = indices.reshape(N // 128, 128)
    jdx2d = flat >> 1
    hb2d = flat & 1

    packed = _sc_gather_pack(jdx2d, hb2d, w32c, N, D)
    return _tc_retile(packed, B, L, D)
